# baseline (speedup 1.0000x reference)
import functools

import jax
import jax.numpy as jnp
from jax import lax
from jax.experimental import pallas as pl
from jax.experimental.pallas import tpu as pltpu
from jax.experimental.pallas import tpu_sc as plsc

_REG = 1e-05
_L = 16


def _sc_gather(ua2, ia2, users, pos, neg, ue2, ie2, B, W):
    info = plsc.get_sparse_core_info()
    nw = info.num_cores * info.num_subcores
    bpw = B // nw

    mesh = plsc.VectorSubcoreMesh(core_axis_name="c", subcore_axis_name="s")

    @functools.partial(
        pl.kernel,
        out_type=tuple(
            jax.ShapeDtypeStruct((B, W), jnp.float32) for _ in range(6)),
        mesh=mesh,
        scratch_types=[
            pltpu.VMEM((bpw,), jnp.int32),
            pltpu.VMEM((bpw,), jnp.int32),
            pltpu.VMEM((bpw,), jnp.int32),
            pltpu.VMEM((bpw, W), jnp.float32),
            pltpu.VMEM((bpw, W), jnp.float32),
            pltpu.VMEM((bpw, W), jnp.float32),
            pltpu.VMEM((bpw, W), jnp.float32),
            pltpu.VMEM((bpw, W), jnp.float32),
            pltpu.VMEM((bpw, W), jnp.float32),
            pltpu.SemaphoreType.DMA,
        ],
    )
    def sc_kernel(ua_h, ia_h, us_h, po_h, ne_h, ue_h, ie_h,
                  u_o, p_o, n_o, up_o, pp_o, np_o,
                  usp_v, pop_v, nep_v,
                  u_v, p_v, n_v, up_v, pp_v, np_v, sem):
        wid = lax.axis_index("s") * info.num_cores + lax.axis_index("c")
        base = wid * bpw

        pltpu.sync_copy(us_h.at[pl.ds(base, bpw)], usp_v)
        pltpu.sync_copy(po_h.at[pl.ds(base, bpw)], pop_v)
        pltpu.sync_copy(ne_h.at[pl.ds(base, bpw)], nep_v)
        for j in range(bpw // _L):
            sl = pl.ds(j * _L, _L)
            usp_v[sl] = lax.shift_right_logical(usp_v[sl], 1)
            pop_v[sl] = lax.shift_right_logical(pop_v[sl], 1)
            nep_v[sl] = lax.shift_right_logical(nep_v[sl], 1)

        copies = [
            pltpu.async_copy(ua_h.at[usp_v], u_v, sem),
            pltpu.async_copy(ia_h.at[pop_v], p_v, sem),
            pltpu.async_copy(ia_h.at[nep_v], n_v, sem),
            pltpu.async_copy(ue_h.at[usp_v], up_v, sem),
            pltpu.async_copy(ie_h.at[pop_v], pp_v, sem),
            pltpu.async_copy(ie_h.at[nep_v], np_v, sem),
        ]
        for c, src, dst in zip(
                copies, (u_v, p_v, n_v, up_v, pp_v, np_v),
                (u_o, p_o, n_o, up_o, pp_o, np_o)):
            c.wait()
            pltpu.sync_copy(src, dst.at[pl.ds(base, bpw)])

    return sc_kernel(ua2, ia2, users, pos, neg, ue2, ie2)


def _tc_finish(u2, p2, n2, up2, pp2, np2, users, pos, neg, D):
    def body(u_ref, p_ref, n_ref, up_ref, pp_ref, np_ref,
             us_ref, po_ref, ne_ref, bpr_ref, emb_ref):
        mu = (us_ref[...] & 1).astype(jnp.float32)
        mp = (po_ref[...] & 1).astype(jnp.float32)
        mn = (ne_ref[...] & 1).astype(jnp.float32)

        u_l, u_r = u_ref[:, :D], u_ref[:, D:]
        p_l, p_r = p_ref[:, :D], p_ref[:, D:]
        n_l, n_r = n_ref[:, :D], n_ref[:, D:]

        def blend2(a_l, a_r, b_l, b_r, ma, mb):
            ll = jnp.sum(a_l * b_l, axis=1)
            lr = jnp.sum(a_l * b_r, axis=1)
            rl = jnp.sum(a_r * b_l, axis=1)
            rr = jnp.sum(a_r * b_r, axis=1)
            na, nb = 1.0 - ma, 1.0 - mb
            return na * nb * ll + na * mb * lr + ma * nb * rl + ma * mb * rr

        s = (blend2(u_l, u_r, p_l, p_r, mu, mp)
             - blend2(u_l, u_r, n_l, n_r, mu, mn))

        def qblend(a_ref, m):
            ql = jnp.sum(a_ref[:, :D] ** 2, axis=1)
            qr = jnp.sum(a_ref[:, D:] ** 2, axis=1)
            return (1.0 - m) * ql + m * qr

        q = qblend(up_ref, mu) + qblend(pp_ref, mp) + qblend(np_ref, mn)

        t = jnp.maximum(-s, 0.0) + jnp.log1p(jnp.exp(-jnp.abs(s)))
        bpr_ref[0, 0] = jnp.sum(t)
        emb_ref[0, 0] = (_REG * 0.5) * jnp.sum(q)

    return pl.pallas_call(
        body,
        out_shape=(
            jax.ShapeDtypeStruct((1, 1), jnp.float32),
            jax.ShapeDtypeStruct((1, 1), jnp.float32),
        ),
        out_specs=(
            pl.BlockSpec(memory_space=pltpu.SMEM),
            pl.BlockSpec(memory_space=pltpu.SMEM),
        ),
    )(u2, p2, n2, up2, pp2, np2, users, pos, neg)


def kernel(ua_embeddings, ia_embeddings, users, pos_items, neg_items,
           user_embedding, item_embedding):
    B = users.shape[0]
    D = ua_embeddings.shape[1]
    W = 2 * D
    users = users.astype(jnp.int32)
    pos_items = pos_items.astype(jnp.int32)
    neg_items = neg_items.astype(jnp.int32)
    gathered = _sc_gather(
        ua_embeddings.reshape(-1, W), ia_embeddings.reshape(-1, W),
        users, pos_items, neg_items,
        user_embedding.reshape(-1, W), item_embedding.reshape(-1, W), B, W)
    bpr, emb = _tc_finish(*gathered, users, pos_items, neg_items, D)
    return (bpr[0, 0], emb[0, 0])


# speedup vs baseline: 1.1300x; 1.1300x over previous
import functools

import jax
import jax.numpy as jnp
from jax import lax
from jax.experimental import pallas as pl
from jax.experimental.pallas import tpu as pltpu
from jax.experimental.pallas import tpu_sc as plsc

_REG = 1e-05
_L = 16


def _sc_partials(ua, ia, users, pos, neg, ue, ie, B, D):
    info = plsc.get_sparse_core_info()
    nw = info.num_cores * info.num_subcores
    bpw = B // nw
    nchunk = D // _L

    mesh = plsc.VectorSubcoreMesh(core_axis_name="c", subcore_axis_name="s")

    @functools.partial(
        pl.kernel,
        out_type=(
            jax.ShapeDtypeStruct((B // 8, 128), jnp.float32),
            jax.ShapeDtypeStruct((nw // 8, 128), jnp.float32),
        ),
        mesh=mesh,
        scratch_types=[
            pltpu.VMEM((bpw,), jnp.int32),
            pltpu.VMEM((bpw,), jnp.int32),
            pltpu.VMEM((bpw,), jnp.int32),
            pltpu.VMEM((bpw, D), jnp.float32),
            pltpu.VMEM((bpw, D), jnp.float32),
            pltpu.VMEM((bpw, D), jnp.float32),
            pltpu.VMEM((bpw, D), jnp.float32),
            pltpu.VMEM((bpw, D), jnp.float32),
            pltpu.VMEM((bpw, D), jnp.float32),
            pltpu.VMEM((bpw // 8, 128), jnp.float32),
            pltpu.VMEM((1, _L), jnp.float32),
            pltpu.SemaphoreType.DMA,
        ],
        compiler_params=pltpu.CompilerParams(use_tc_tiling_on_sc=False),
    )
    def sc_kernel(ua_h, ia_h, us_h, po_h, ne_h, ue_h, ie_h,
                  dot_h, reg_h,
                  us_v, po_v, ne_v, u_v, p_v, n_v, up_v, pp_v, np_v,
                  dot_v, reg_v, sem):
        wid = lax.axis_index("s") * info.num_cores + lax.axis_index("c")
        base = wid * bpw

        pltpu.sync_copy(us_h.at[pl.ds(base, bpw)], us_v)
        pltpu.sync_copy(po_h.at[pl.ds(base, bpw)], po_v)
        pltpu.sync_copy(ne_h.at[pl.ds(base, bpw)], ne_v)

        copies = [
            pltpu.async_copy(ua_h.at[us_v], u_v, sem),
            pltpu.async_copy(ia_h.at[po_v], p_v, sem),
            pltpu.async_copy(ia_h.at[ne_v], n_v, sem),
            pltpu.async_copy(ue_h.at[us_v], up_v, sem),
            pltpu.async_copy(ie_h.at[po_v], pp_v, sem),
            pltpu.async_copy(ie_h.at[ne_v], np_v, sem),
        ]
        for c in copies:
            c.wait()

        acc_reg = jnp.zeros((_L,), jnp.float32)
        for sub in range(8):
            slot = pl.ds(sub * _L, _L)

            def g_body(g, acc, sub=sub, slot=slot):
                r = g * 8 + sub
                d_acc = jnp.zeros((_L,), jnp.float32)
                for k in range(nchunk):
                    sl = pl.ds(k * _L, _L)
                    d_acc = d_acc + u_v[r, sl] * (p_v[r, sl] - n_v[r, sl])
                    au = up_v[r, sl]
                    ap = pp_v[r, sl]
                    an = np_v[r, sl]
                    acc = acc + au * au + ap * ap + an * an
                dot_v[g, slot] = d_acc
                return acc

            acc_reg = lax.fori_loop(0, bpw // 8, g_body, acc_reg)
        reg_v[0, :] = acc_reg

        pltpu.sync_copy(dot_v, dot_h.at[pl.ds(wid * (bpw // 8), bpw // 8)])
        pltpu.sync_copy(
            reg_v,
            reg_h.at[pl.ds(wid // 8, 1), pl.ds((wid % 8) * _L, _L)])

    return sc_kernel(ua, ia, users, pos, neg, ue, ie)


def _tc_finish(dot_part, reg_part):
    def body(dot_ref, reg_ref, bpr_ref, emb_ref):
        part = dot_ref[...]
        j = lax.broadcasted_iota(jnp.int32, (128, 8), 0)
        a = lax.broadcasted_iota(jnp.int32, (128, 8), 1)
        sel = (j // _L == a).astype(jnp.float32)
        s = jnp.dot(part, sel, preferred_element_type=jnp.float32)
        t = jnp.maximum(-s, 0.0) + jnp.log1p(jnp.exp(-jnp.abs(s)))
        bpr_ref[0, 0] = jnp.sum(t)
        emb_ref[0, 0] = (_REG * 0.5) * jnp.sum(reg_ref[...])

    return pl.pallas_call(
        body,
        out_shape=(
            jax.ShapeDtypeStruct((1, 1), jnp.float32),
            jax.ShapeDtypeStruct((1, 1), jnp.float32),
        ),
        out_specs=(
            pl.BlockSpec(memory_space=pltpu.SMEM),
            pl.BlockSpec(memory_space=pltpu.SMEM),
        ),
    )(dot_part, reg_part)


def kernel(ua_embeddings, ia_embeddings, users, pos_items, neg_items,
           user_embedding, item_embedding):
    B = users.shape[0]
    D = ua_embeddings.shape[1]
    dot_part, reg_part = _sc_partials(
        ua_embeddings, ia_embeddings,
        users.astype(jnp.int32), pos_items.astype(jnp.int32),
        neg_items.astype(jnp.int32),
        user_embedding, item_embedding, B, D)
    bpr, emb = _tc_finish(dot_part, reg_part)
    return (bpr[0, 0], emb[0, 0])


# speedup vs baseline: 3.4959x; 3.0936x over previous
import functools

import jax
import jax.numpy as jnp
from jax import lax
from jax.experimental import pallas as pl
from jax.experimental.pallas import tpu as pltpu
from jax.experimental.pallas import tpu_sc as plsc

_REG = 1e-05
_L = 16


def _sc_partials(ua_t, ia_t, users, pos, neg, ue_t, ie_t, B, D, N):
    info = plsc.get_sparse_core_info()
    nw = info.num_cores * info.num_subcores
    fpw = D // nw
    nb = B // _L

    mesh = plsc.VectorSubcoreMesh(core_axis_name="c", subcore_axis_name="s")

    @functools.partial(
        pl.kernel,
        out_type=(
            jax.ShapeDtypeStruct((nw, B), jnp.float32),
            jax.ShapeDtypeStruct((nw, 128), jnp.float32),
        ),
        mesh=mesh,
        scratch_types=[
            pltpu.VMEM((B,), jnp.int32),
            pltpu.VMEM((B,), jnp.int32),
            pltpu.VMEM((B,), jnp.int32),
            pltpu.VMEM((N,), jnp.float32),
            pltpu.VMEM((B,), jnp.float32),
            pltpu.VMEM((B,), jnp.float32),
            pltpu.VMEM((1, 128), jnp.float32),
            pltpu.SemaphoreType.DMA,
        ],
        compiler_params=pltpu.CompilerParams(needs_layout_passes=False),
    )
    def sc_kernel(ua_h, ia_h, us_h, po_h, ne_h, ue_h, ie_h,
                  dot_h, reg_h,
                  us_v, po_v, ne_v, col_v, uval_v, dacc_v, reg_v, sem):
        wid = lax.axis_index("s") * info.num_cores + lax.axis_index("c")

        pltpu.sync_copy(us_h, us_v)
        pltpu.sync_copy(po_h, po_v)
        pltpu.sync_copy(ne_h, ne_v)

        def zero_body(i, _):
            dacc_v[pl.ds(i * _L, _L)] = jnp.zeros((_L,), jnp.float32)
            return 0

        lax.fori_loop(0, nb, zero_body, 0)

        acc_reg = jnp.zeros((_L,), jnp.float32)
        for jj in range(fpw):
            fj = wid * fpw + jj

            pltpu.sync_copy(ua_h.at[fj], col_v)

            def gu_body(i, _):
                sl = pl.ds(i * _L, _L)
                uval_v[sl] = plsc.load_gather(col_v, [us_v[sl]])
                return 0

            lax.fori_loop(0, nb, gu_body, 0)

            pltpu.sync_copy(ia_h.at[fj], col_v)

            def dot_body(i, _):
                sl = pl.ds(i * _L, _L)
                pv = plsc.load_gather(col_v, [po_v[sl]])
                nv = plsc.load_gather(col_v, [ne_v[sl]])
                dacc_v[sl] = dacc_v[sl] + uval_v[sl] * (pv - nv)
                return 0

            lax.fori_loop(0, nb, dot_body, 0)

            pltpu.sync_copy(ue_h.at[fj], col_v)

            def qu_body(i, acc):
                w = plsc.load_gather(col_v, [us_v[pl.ds(i * _L, _L)]])
                return acc + w * w

            acc_reg = lax.fori_loop(0, nb, qu_body, acc_reg)

            pltpu.sync_copy(ie_h.at[fj], col_v)

            def qi_body(i, acc):
                sl = pl.ds(i * _L, _L)
                wp = plsc.load_gather(col_v, [po_v[sl]])
                wn = plsc.load_gather(col_v, [ne_v[sl]])
                return acc + wp * wp + wn * wn

            acc_reg = lax.fori_loop(0, nb, qi_body, acc_reg)

        for z in range(128 // _L):
            reg_v[0, pl.ds(z * _L, _L)] = jnp.zeros((_L,), jnp.float32)
        reg_v[0, pl.ds(0, _L)] = acc_reg
        pltpu.sync_copy(dacc_v, dot_h.at[wid])
        pltpu.sync_copy(reg_v, reg_h.at[pl.ds(wid, 1)])

    return sc_kernel(ua_t, ia_t, users, pos, neg, ue_t, ie_t)


def _tc_finish(dot_part, reg_part):
    def body(dot_ref, reg_ref, bpr_ref, emb_ref):
        s = jnp.sum(dot_ref[...], axis=0)
        t = jnp.maximum(-s, 0.0) + jnp.log1p(jnp.exp(-jnp.abs(s)))
        bpr_ref[0, 0] = jnp.sum(t)
        emb_ref[0, 0] = (_REG * 0.5) * jnp.sum(reg_ref[...])

    return pl.pallas_call(
        body,
        out_shape=(
            jax.ShapeDtypeStruct((1, 1), jnp.float32),
            jax.ShapeDtypeStruct((1, 1), jnp.float32),
        ),
        out_specs=(
            pl.BlockSpec(memory_space=pltpu.SMEM),
            pl.BlockSpec(memory_space=pltpu.SMEM),
        ),
    )(dot_part, reg_part)


def kernel(ua_embeddings, ia_embeddings, users, pos_items, neg_items,
           user_embedding, item_embedding):
    B = users.shape[0]
    N, D = ua_embeddings.shape
    dot_part, reg_part = _sc_partials(
        ua_embeddings.T, ia_embeddings.T,
        users.astype(jnp.int32), pos_items.astype(jnp.int32),
        neg_items.astype(jnp.int32),
        user_embedding.T, item_embedding.T, B, D, N)
    bpr, emb = _tc_finish(dot_part, reg_part)
    return (bpr[0, 0], emb[0, 0])
